# baseline (speedup 1.0000x reference)
import functools

import jax
import jax.numpy as jnp
from jax import lax
from jax.experimental import pallas as pl
from jax.experimental.pallas import tpu as pltpu
from jax.experimental.pallas import tpu_sc as plsc

F32 = jnp.float32
I32 = jnp.int32
G = 128
INNER = 4
SLOTS = 2 * INNER


def _sc_geom():
    try:
        info = plsc.get_sparse_core_info()
        return int(info.num_cores), int(info.num_subcores)
    except Exception:
        return 2, 16


def _npad(n, ns):
    q = 8 * ns
    return q * (-(-(n + 1) // q))


def _pad_edges(src, dst, n, nw):
    e = src.shape[0]
    quantum = nw * G * SLOTS
    epad = quantum * (-(-e // quantum))
    pad = epad - e
    src_p = jnp.concatenate([src, jnp.zeros((pad,), I32)])
    dst_p = jnp.concatenate([dst, jnp.full((pad,), n, I32)])
    return src_p.reshape(-1, G), dst_p.reshape(-1, G)


def _segsum(u, src3, dst3, npad):
    n, c = u.shape
    nc, ns = _sc_geom()
    nw = nc * ns
    ng = src3.shape[0] // nw
    outer = ng // SLOTS
    rpt = npad // ns
    zeros = jnp.zeros((npad, c), F32)
    mesh = plsc.VectorSubcoreMesh(
        core_axis_name="c", subcore_axis_name="s",
        num_cores=nc, num_subcores=ns)

    @functools.partial(
        pl.kernel,
        out_type=jax.ShapeDtypeStruct((nc, npad, c), F32),
        mesh=mesh,
        scratch_types=[
            pltpu.VMEM((ng, G), I32),
            pltpu.VMEM((ng, G), I32),
            pltpu.VMEM((SLOTS, G, c), F32),
        ] + [pltpu.SemaphoreType.DMA] * (2 * SLOTS) + [
            pltpu.VMEM_SHARED((npad, c), F32),
        ],
        compiler_params=pltpu.CompilerParams(use_tc_tiling_on_sc=False),
    )
    def k(u_hbm, src_hbm, dst_hbm, z_hbm, out_hbm,
          src_v, dst_v, rows_v, *rest):
        gsem = rest[:SLOTS]
        ssem = rest[SLOTS:2 * SLOTS]
        acc = rest[2 * SLOTS]
        ci = lax.axis_index("c")
        si = lax.axis_index("s")
        tid = ci * ns + si
        pltpu.sync_copy(z_hbm.at[pl.ds(si * rpt, rpt)],
                        acc.at[pl.ds(si * rpt, rpt)])
        plsc.subcore_barrier()
        g0 = tid * ng
        pltpu.sync_copy(src_hbm.at[pl.ds(g0, ng)], src_v)
        pltpu.sync_copy(dst_hbm.at[pl.ds(g0, ng)], dst_v)
        for b in range(INNER):
            pltpu.async_copy(u_hbm.at[src_v.at[b]], rows_v.at[b], gsem[b])

        def obody(o, carry):
            for j in range(SLOTS):
                g = o * SLOTS + j
                b2 = (j + INNER) % SLOTS
                if j < INNER:
                    @pl.when(o > 0)
                    def _drain():
                        pltpu.make_async_copy(
                            rows_v.at[b2], acc.at[dst_v.at[0]],
                            ssem[b2]).wait()
                    pltpu.async_copy(
                        u_hbm.at[src_v.at[g + INNER]], rows_v.at[b2],
                        gsem[b2])
                else:
                    pltpu.make_async_copy(
                        rows_v.at[b2], acc.at[dst_v.at[0]], ssem[b2]).wait()

                    @pl.when(o < outer - 1)
                    def _fire():
                        pltpu.async_copy(
                            u_hbm.at[src_v.at[g + INNER]], rows_v.at[b2],
                            gsem[b2])
                pltpu.make_async_copy(
                    u_hbm.at[src_v.at[g]], rows_v.at[j], gsem[j]).wait()
                pltpu.async_copy(rows_v.at[j], acc.at[dst_v.at[g]], ssem[j],
                                 add=True)
            return carry

        lax.fori_loop(0, outer, obody, 0)
        for j in range(INNER, SLOTS):
            pltpu.make_async_copy(
                rows_v.at[j], acc.at[dst_v.at[0]], ssem[j]).wait()
        plsc.subcore_barrier()
        pltpu.sync_copy(acc.at[pl.ds(si * rpt, rpt)],
                        out_hbm.at[ci, pl.ds(si * rpt, rpt)])

    return k(u, src3, dst3, zeros)


_DEGW = 8


def _deg(dst3, npad):
    nc, ns = _sc_geom()
    nw = nc * ns
    ng = dst3.shape[0] // nw
    rpt = npad // ns
    ones = jnp.ones((G, _DEGW), F32)
    zeros = jnp.zeros((npad, _DEGW), F32)
    mesh = plsc.VectorSubcoreMesh(
        core_axis_name="c", subcore_axis_name="s",
        num_cores=nc, num_subcores=ns)

    @functools.partial(
        pl.kernel,
        out_type=jax.ShapeDtypeStruct((nc, npad, _DEGW), F32),
        mesh=mesh,
        scratch_types=[
            pltpu.VMEM((ng, G), I32),
            pltpu.VMEM((G, _DEGW), F32),
            pltpu.VMEM_SHARED((npad, _DEGW), F32),
        ] + [pltpu.SemaphoreType.DMA] * SLOTS,
        compiler_params=pltpu.CompilerParams(use_tc_tiling_on_sc=False),
    )
    def k(dst_hbm, ones_hbm, z_hbm, out_hbm, dst_v, ones_v, acc, *ssem):
        ci = lax.axis_index("c")
        si = lax.axis_index("s")
        tid = ci * ns + si
        pltpu.sync_copy(z_hbm.at[pl.ds(si * rpt, rpt)],
                        acc.at[pl.ds(si * rpt, rpt)])
        plsc.subcore_barrier()
        pltpu.sync_copy(dst_hbm.at[pl.ds(tid * ng, ng)], dst_v)
        pltpu.sync_copy(ones_hbm.at[pl.ds(0, G)], ones_v.at[pl.ds(0, G)])
        outer = ng // SLOTS

        def obody(o, carry):
            for j in range(SLOTS):
                g = o * SLOTS + j

                @pl.when(o > 0)
                def _drain():
                    pltpu.make_async_copy(
                        ones_v, acc.at[dst_v.at[0]], ssem[j]).wait()
                pltpu.async_copy(ones_v, acc.at[dst_v.at[g]], ssem[j],
                                 add=True)
            return carry

        lax.fori_loop(0, outer, obody, 0)
        for j in range(SLOTS):
            pltpu.make_async_copy(
                ones_v, acc.at[dst_v.at[0]], ssem[j]).wait()
        plsc.subcore_barrier()
        pltpu.sync_copy(acc.at[pl.ds(si * rpt, rpt)],
                        out_hbm.at[ci, pl.ds(si * rpt, rpt)])

    return k(dst3, ones, zeros)


_BLK = 2048


def _prep(p0, p1, a0, a1, scale):
    n, c = a0.shape
    nb = -(-n // _BLK)

    def body(p0_r, p1_r, a0_r, a1_r, dis_r, t0_r, u0_r):
        deg = p0_r[...] + p1_r[...]
        dis = jnp.where(deg > 0, lax.rsqrt(deg), 0.0)
        t0 = scale * (a0_r[...] + a1_r[...])
        dis_r[...] = dis
        t0_r[...] = t0
        u0_r[...] = dis * t0

    return pl.pallas_call(
        body,
        grid=(nb,),
        in_specs=[
            pl.BlockSpec((_BLK, 1), lambda i: (i, 0)),
            pl.BlockSpec((_BLK, 1), lambda i: (i, 0)),
            pl.BlockSpec((_BLK, c), lambda i: (i, 0)),
            pl.BlockSpec((_BLK, c), lambda i: (i, 0)),
        ],
        out_specs=[
            pl.BlockSpec((_BLK, 1), lambda i: (i, 0)),
            pl.BlockSpec((_BLK, c), lambda i: (i, 0)),
            pl.BlockSpec((_BLK, c), lambda i: (i, 0)),
        ],
        out_shape=[
            jax.ShapeDtypeStruct((n, 1), F32),
            jax.ShapeDtypeStruct((n, c), F32),
            jax.ShapeDtypeStruct((n, c), F32),
        ],
    )(p0, p1, a0, a1)


def _combine(s0, s1, dis, tprev, alpha, beta):
    n, c = s0.shape
    nb = -(-n // _BLK)

    def body(s0_r, s1_r, dis_r, tp_r, t_r, u_r):
        t = alpha * (dis_r[...] * (s0_r[...] + s1_r[...])) + beta * tp_r[...]
        t_r[...] = t
        u_r[...] = dis_r[...] * t

    return pl.pallas_call(
        body,
        grid=(nb,),
        in_specs=[
            pl.BlockSpec((_BLK, c), lambda i: (i, 0)),
            pl.BlockSpec((_BLK, c), lambda i: (i, 0)),
            pl.BlockSpec((_BLK, 1), lambda i: (i, 0)),
            pl.BlockSpec((_BLK, c), lambda i: (i, 0)),
        ],
        out_specs=[
            pl.BlockSpec((_BLK, c), lambda i: (i, 0)),
            pl.BlockSpec((_BLK, c), lambda i: (i, 0)),
        ],
        out_shape=[
            jax.ShapeDtypeStruct((n, c), F32),
            jax.ShapeDtypeStruct((n, c), F32),
        ],
    )(s0, s1, dis, tprev)


def _cheb_matmul(ts, w, b, relu):
    n, c = ts[0].shape
    kk, _, cout = w.shape
    nb = -(-n // _BLK)

    def body(*refs):
        t_refs = refs[:kk]
        w_r, b_r, h_r = refs[kk], refs[kk + 1], refs[kk + 2]
        acc = b_r[...].astype(F32) + jnp.zeros((_BLK, cout), F32)
        for k in range(kk):
            acc = acc + jnp.dot(t_refs[k][...], w_r[k],
                                preferred_element_type=F32)
        if relu:
            acc = jnp.maximum(acc, 0.0)
        h_r[...] = acc

    return pl.pallas_call(
        body,
        grid=(nb,),
        in_specs=[pl.BlockSpec((_BLK, c), lambda i: (i, 0))
                  for _ in range(kk)] + [
            pl.BlockSpec((kk, c, cout), lambda i: (0, 0, 0)),
            pl.BlockSpec((1, cout), lambda i: (0, 0)),
        ],
        out_specs=pl.BlockSpec((_BLK, cout), lambda i: (i, 0)),
        out_shape=jax.ShapeDtypeStruct((n, cout), F32),
    )(*ts, w, b.reshape(1, cout))


def _matvec(wlin, hflat, blin):
    ncls, kdim = wlin.shape
    cb = 8
    ni = ncls // cb

    def body(h_r, w_r, b_r, z_r):
        z_r[...] = b_r[...] + lax.dot_general(
            w_r[...], h_r[...], (((1,), (1,)), ((), ())),
            preferred_element_type=F32)

    out = pl.pallas_call(
        body,
        grid=(ni,),
        in_specs=[
            pl.BlockSpec((1, kdim), lambda i: (0, 0)),
            pl.BlockSpec((cb, kdim), lambda i: (i, 0)),
            pl.BlockSpec((cb, 1), lambda i: (i, 0)),
        ],
        out_specs=pl.BlockSpec((cb, 1), lambda i: (i, 0)),
        out_shape=jax.ShapeDtypeStruct((ncls, 1), F32),
    )(hflat.reshape(1, kdim), wlin, blin.reshape(ncls, 1))
    return out.reshape(ncls)


def _cheb_layer(a0, a1, scale, ei, w, b, n, relu):
    nc, ns = _sc_geom()
    nw = nc * ns
    src3, dst3 = _pad_edges(ei[0], ei[1], n, nw)
    npad = _npad(n, ns)
    degp = _deg(dst3, npad)
    dis, t0, u = _prep(degp[0, :n, :1], degp[1, :n, :1], a0, a1, scale)
    ts = [t0]
    kk = w.shape[0]
    for k in range(1, kk):
        sp = _segsum(u, src3, dst3, npad)
        alpha, beta = (-1.0, 0.0) if k == 1 else (-2.0, -1.0)
        tprev = ts[k - 2] if k >= 2 else t0
        t, u = _combine(sp[0, :n], sp[1, :n], dis, tprev, alpha, beta)
        ts.append(t)
    return _cheb_matmul(ts, w, b, relu)


def _pool(h, rows, cols, n_out):
    nc, ns = _sc_geom()
    src3, dst3 = _pad_edges(cols, rows, n_out, nc * ns)
    npad = _npad(n_out, ns)
    sp = _segsum(h, src3, dst3, npad)
    return sp[0, :n_out], sp[1, :n_out]


def kernel(x, ei0, ei1, ei2, d0_rows, d0_cols, d0_vals,
           d1_rows, d1_cols, d1_vals,
           W0, b0, W1, b1, W2, b2, Wlin, blin):
    n0, n1, n2 = 50000, 12500, 3125
    x8 = jnp.pad(x, ((0, 0), (0, 5)))
    w0p = jnp.pad(W0, ((0, 0), (0, 5), (0, 0)))
    h0 = _cheb_layer(x8, x8, 0.5, ei0, w0p, b0, n0, relu=True)
    a0, a1 = _pool(h0, d0_rows, d0_cols, n1)
    h1 = _cheb_layer(a0, a1, 0.25, ei1, W1, b1, n1, relu=True)
    a0, a1 = _pool(h1, d1_rows, d1_cols, n2)
    h2 = _cheb_layer(a0, a1, 0.25, ei2, W2, b2, n2, relu=False)
    return _matvec(Wlin, h2.reshape(-1), blin)


# speedup vs baseline: 1.2586x; 1.2586x over previous
import functools

import jax
import jax.numpy as jnp
from jax import lax
from jax.experimental import pallas as pl
from jax.experimental.pallas import tpu as pltpu
from jax.experimental.pallas import tpu_sc as plsc

F32 = jnp.float32
I32 = jnp.int32
G = 128
INNER = 4


def _sc_geom():
    try:
        info = plsc.get_sparse_core_info()
        return int(info.num_cores), int(info.num_subcores)
    except Exception:
        return 2, 16


def _npad(n, ns):
    q = 8 * ns
    return q * (-(-(n + 1) // q))


def _pad_edges(src, dst, n, nw):
    e = src.shape[0]
    quantum = nw * G * INNER
    epad = quantum * (-(-e // quantum))
    pad = epad - e
    src_p = jnp.concatenate([src, jnp.zeros((pad,), I32)])
    dst_p = jnp.concatenate([dst, jnp.full((pad,), n, I32)])
    return src_p.reshape(-1, G), dst_p.reshape(-1, G)


def _segsum(u, src3, dst3, npad):
    n, c = u.shape
    nc, ns = _sc_geom()
    nw = nc * ns
    ng = src3.shape[0] // nw
    outer = ng // INNER
    rpt = npad // ns
    zeros = jnp.zeros((npad, c), F32)
    mesh = plsc.VectorSubcoreMesh(
        core_axis_name="c", subcore_axis_name="s",
        num_cores=nc, num_subcores=ns)

    @functools.partial(
        pl.kernel,
        out_type=jax.ShapeDtypeStruct((nc, npad, c), F32),
        mesh=mesh,
        scratch_types=[
            pltpu.VMEM((ng, G), I32),
            pltpu.VMEM((ng, G), I32),
            pltpu.VMEM((INNER, G, c), F32),
        ] + [pltpu.SemaphoreType.DMA] * INNER + [
            pltpu.VMEM_SHARED((npad, c), F32),
        ],
        compiler_params=pltpu.CompilerParams(use_tc_tiling_on_sc=False),
    )
    def k(u_hbm, src_hbm, dst_hbm, z_hbm, out_hbm,
          src_v, dst_v, rows_v, *rest):
        sems = rest[:INNER]
        acc = rest[INNER]
        ci = lax.axis_index("c")
        si = lax.axis_index("s")
        tid = ci * ns + si
        pltpu.sync_copy(z_hbm.at[pl.ds(si * rpt, rpt)],
                        acc.at[pl.ds(si * rpt, rpt)])
        plsc.subcore_barrier()
        g0 = tid * ng
        pltpu.sync_copy(src_hbm.at[pl.ds(g0, ng)], src_v)
        pltpu.sync_copy(dst_hbm.at[pl.ds(g0, ng)], dst_v)
        for b in range(INNER):
            pltpu.async_copy(u_hbm.at[src_v.at[b]], rows_v.at[b], sems[b])

        def obody(o, carry):
            for b in range(INNER):
                g = o * INNER + b
                pltpu.make_async_copy(
                    u_hbm.at[src_v.at[g]], rows_v.at[b], sems[b]).wait()
                pltpu.sync_copy(rows_v.at[b], acc.at[dst_v.at[g]], add=True)

                @pl.when(o < outer - 1)
                def _fire():
                    pltpu.async_copy(
                        u_hbm.at[src_v.at[g + INNER]], rows_v.at[b], sems[b])
            return carry

        lax.fori_loop(0, outer, obody, 0)
        plsc.subcore_barrier()
        pltpu.sync_copy(acc.at[pl.ds(si * rpt, rpt)],
                        out_hbm.at[ci, pl.ds(si * rpt, rpt)])

    return k(u, src3, dst3, zeros)


_DEGW = 8


def _deg3(dst3s, npads):
    nc, ns = _sc_geom()
    nw = nc * ns
    ngs = [d.shape[0] // nw for d in dst3s]
    rpts = [np_ // ns for np_ in npads]
    ones = jnp.ones((G, _DEGW), F32)
    zeros = jnp.zeros((max(npads), _DEGW), F32)
    mesh = plsc.VectorSubcoreMesh(
        core_axis_name="c", subcore_axis_name="s",
        num_cores=nc, num_subcores=ns)

    @functools.partial(
        pl.kernel,
        out_type=[jax.ShapeDtypeStruct((nc, np_, _DEGW), F32)
                  for np_ in npads],
        mesh=mesh,
        scratch_types=[pltpu.VMEM((ng, G), I32) for ng in ngs] + [
            pltpu.VMEM((G, _DEGW), F32),
        ] + [pltpu.VMEM_SHARED((np_, _DEGW), F32) for np_ in npads],
        compiler_params=pltpu.CompilerParams(use_tc_tiling_on_sc=False),
    )
    def k(d0_hbm, d1_hbm, d2_hbm, ones_hbm, z_hbm, o0_hbm, o1_hbm, o2_hbm,
          v0, v1, v2, ones_v, a0, a1, a2):
        dhbm, dv, accs, outs = ([d0_hbm, d1_hbm, d2_hbm], [v0, v1, v2],
                                [a0, a1, a2], [o0_hbm, o1_hbm, o2_hbm])
        ci = lax.axis_index("c")
        si = lax.axis_index("s")
        tid = ci * ns + si
        for l in range(3):
            pltpu.sync_copy(z_hbm.at[pl.ds(si * rpts[l], rpts[l])],
                            accs[l].at[pl.ds(si * rpts[l], rpts[l])])
        plsc.subcore_barrier()
        pltpu.sync_copy(ones_hbm.at[pl.ds(0, G)], ones_v.at[pl.ds(0, G)])
        for l in range(3):
            pltpu.sync_copy(dhbm[l].at[pl.ds(tid * ngs[l], ngs[l])], dv[l])

            def obody(g, carry, l=l):
                pltpu.sync_copy(ones_v, accs[l].at[dv[l].at[g]], add=True)
                return carry

            lax.fori_loop(0, ngs[l], obody, 0)
        plsc.subcore_barrier()
        for l in range(3):
            pltpu.sync_copy(accs[l].at[pl.ds(si * rpts[l], rpts[l])],
                            outs[l].at[ci, pl.ds(si * rpts[l], rpts[l])])

    return k(dst3s[0], dst3s[1], dst3s[2], ones, zeros)


_BLK = 2048


def _prep(p0, p1, a0, a1, scale):
    n, c = a0.shape
    nb = -(-n // _BLK)

    def body(p0_r, p1_r, a0_r, a1_r, dis_r, t0_r, u0_r):
        deg = p0_r[...] + p1_r[...]
        dis = jnp.where(deg > 0, lax.rsqrt(deg), 0.0)
        t0 = scale * (a0_r[...] + a1_r[...])
        dis_r[...] = dis
        t0_r[...] = t0
        u0_r[...] = dis * t0

    return pl.pallas_call(
        body,
        grid=(nb,),
        in_specs=[
            pl.BlockSpec((_BLK, 1), lambda i: (i, 0)),
            pl.BlockSpec((_BLK, 1), lambda i: (i, 0)),
            pl.BlockSpec((_BLK, c), lambda i: (i, 0)),
            pl.BlockSpec((_BLK, c), lambda i: (i, 0)),
        ],
        out_specs=[
            pl.BlockSpec((_BLK, 1), lambda i: (i, 0)),
            pl.BlockSpec((_BLK, c), lambda i: (i, 0)),
            pl.BlockSpec((_BLK, c), lambda i: (i, 0)),
        ],
        out_shape=[
            jax.ShapeDtypeStruct((n, 1), F32),
            jax.ShapeDtypeStruct((n, c), F32),
            jax.ShapeDtypeStruct((n, c), F32),
        ],
    )(p0, p1, a0, a1)


def _combine(s0, s1, dis, tprev, alpha, beta):
    n, c = s0.shape
    nb = -(-n // _BLK)

    def body(s0_r, s1_r, dis_r, tp_r, t_r, u_r):
        t = alpha * (dis_r[...] * (s0_r[...] + s1_r[...])) + beta * tp_r[...]
        t_r[...] = t
        u_r[...] = dis_r[...] * t

    return pl.pallas_call(
        body,
        grid=(nb,),
        in_specs=[
            pl.BlockSpec((_BLK, c), lambda i: (i, 0)),
            pl.BlockSpec((_BLK, c), lambda i: (i, 0)),
            pl.BlockSpec((_BLK, 1), lambda i: (i, 0)),
            pl.BlockSpec((_BLK, c), lambda i: (i, 0)),
        ],
        out_specs=[
            pl.BlockSpec((_BLK, c), lambda i: (i, 0)),
            pl.BlockSpec((_BLK, c), lambda i: (i, 0)),
        ],
        out_shape=[
            jax.ShapeDtypeStruct((n, c), F32),
            jax.ShapeDtypeStruct((n, c), F32),
        ],
    )(s0, s1, dis, tprev)


def _cheb_matmul(ts, w, b, relu):
    n, c = ts[0].shape
    kk, _, cout = w.shape
    nb = -(-n // _BLK)

    def body(*refs):
        t_refs = refs[:kk]
        w_r, b_r, h_r = refs[kk], refs[kk + 1], refs[kk + 2]
        acc = b_r[...].astype(F32) + jnp.zeros((_BLK, cout), F32)
        for k in range(kk):
            acc = acc + jnp.dot(t_refs[k][...], w_r[k],
                                preferred_element_type=F32)
        if relu:
            acc = jnp.maximum(acc, 0.0)
        h_r[...] = acc

    return pl.pallas_call(
        body,
        grid=(nb,),
        in_specs=[pl.BlockSpec((_BLK, c), lambda i: (i, 0))
                  for _ in range(kk)] + [
            pl.BlockSpec((kk, c, cout), lambda i: (0, 0, 0)),
            pl.BlockSpec((1, cout), lambda i: (0, 0)),
        ],
        out_specs=pl.BlockSpec((_BLK, cout), lambda i: (i, 0)),
        out_shape=jax.ShapeDtypeStruct((n, cout), F32),
    )(*ts, w, b.reshape(1, cout))


def _matvec(wlin, hflat, blin):
    ncls, kdim = wlin.shape
    cb = 8
    ni = ncls // cb

    def body(h_r, w_r, b_r, z_r):
        z_r[...] = b_r[...] + lax.dot_general(
            w_r[...], h_r[...], (((1,), (1,)), ((), ())),
            preferred_element_type=F32)

    out = pl.pallas_call(
        body,
        grid=(ni,),
        in_specs=[
            pl.BlockSpec((1, kdim), lambda i: (0, 0)),
            pl.BlockSpec((cb, kdim), lambda i: (i, 0)),
            pl.BlockSpec((cb, 1), lambda i: (i, 0)),
        ],
        out_specs=pl.BlockSpec((cb, 1), lambda i: (i, 0)),
        out_shape=jax.ShapeDtypeStruct((ncls, 1), F32),
    )(hflat.reshape(1, kdim), wlin, blin.reshape(ncls, 1))
    return out.reshape(ncls)


def _cheb_layer(a0, a1, scale, src3, dst3, degp, w, b, n, relu):
    nc, ns = _sc_geom()
    npad = _npad(n, ns)
    dis, t0, u = _prep(degp[0, :n, :1], degp[1, :n, :1], a0, a1, scale)
    ts = [t0]
    kk = w.shape[0]
    for k in range(1, kk):
        sp = _segsum(u, src3, dst3, npad)
        alpha, beta = (-1.0, 0.0) if k == 1 else (-2.0, -1.0)
        tprev = ts[k - 2] if k >= 2 else t0
        t, u = _combine(sp[0, :n], sp[1, :n], dis, tprev, alpha, beta)
        ts.append(t)
    return _cheb_matmul(ts, w, b, relu)


def _pool(h, rows, cols, n_out):
    nc, ns = _sc_geom()
    src3, dst3 = _pad_edges(cols, rows, n_out, nc * ns)
    npad = _npad(n_out, ns)
    sp = _segsum(h, src3, dst3, npad)
    return sp[0, :n_out], sp[1, :n_out]


def kernel(x, ei0, ei1, ei2, d0_rows, d0_cols, d0_vals,
           d1_rows, d1_cols, d1_vals,
           W0, b0, W1, b1, W2, b2, Wlin, blin):
    n0, n1, n2 = 50000, 12500, 3125
    x8 = jnp.pad(x, ((0, 0), (0, 5)))
    w0p = jnp.pad(W0, ((0, 0), (0, 5), (0, 0)))
    nc, ns = _sc_geom()
    nw = nc * ns
    s0, d0 = _pad_edges(ei0[0], ei0[1], n0, nw)
    s1, d1 = _pad_edges(ei1[0], ei1[1], n1, nw)
    s2, d2 = _pad_edges(ei2[0], ei2[1], n2, nw)
    npads = [_npad(n, ns) for n in (n0, n1, n2)]
    degs = _deg3([d0, d1, d2], npads)
    h0 = _cheb_layer(x8, x8, 0.5, s0, d0, degs[0], w0p, b0, n0, relu=True)
    a0, a1 = _pool(h0, d0_rows, d0_cols, n1)
    h1 = _cheb_layer(a0, a1, 0.25, s1, d1, degs[1], W1, b1, n1, relu=True)
    a0, a1 = _pool(h1, d1_rows, d1_cols, n2)
    h2 = _cheb_layer(a0, a1, 0.25, s2, d2, degs[2], W2, b2, n2, relu=False)
    return _matvec(Wlin, h2.reshape(-1), blin)


# speedup vs baseline: 1.4803x; 1.1762x over previous
import functools

import jax
import jax.numpy as jnp
from jax import lax
from jax.experimental import pallas as pl
from jax.experimental.pallas import tpu as pltpu
from jax.experimental.pallas import tpu_sc as plsc

F32 = jnp.float32
I32 = jnp.int32
G = 128
INNER = 4


def _sc_geom():
    try:
        info = plsc.get_sparse_core_info()
        return int(info.num_cores), int(info.num_subcores)
    except Exception:
        return 2, 16


def _npad(n, ns):
    q = 8 * ns
    return q * (-(-(n + 1) // q))


def _pad_edges(src, dst, n, nw):
    e = src.shape[0]
    quantum = nw * G * INNER
    epad = quantum * (-(-e // quantum))
    pad = epad - e
    src_p = jnp.concatenate([src, jnp.zeros((pad,), I32)])
    dst_p = jnp.concatenate([dst, jnp.full((pad,), n, I32)])
    return src_p.reshape(-1, G), dst_p.reshape(-1, G)


def _segsum(u, src3, dst3, npad):
    n, c = u.shape
    nc, ns = _sc_geom()
    nw = nc * ns
    ng = src3.shape[0] // nw
    outer = ng // INNER
    rpt = npad // ns
    zeros = jnp.zeros((npad, c), F32)
    mesh = plsc.VectorSubcoreMesh(
        core_axis_name="c", subcore_axis_name="s",
        num_cores=nc, num_subcores=ns)

    @functools.partial(
        pl.kernel,
        out_type=jax.ShapeDtypeStruct((nc, npad, c), F32),
        mesh=mesh,
        scratch_types=[
            pltpu.VMEM((ng, G), I32),
            pltpu.VMEM((ng, G), I32),
            pltpu.VMEM((INNER, G, c), F32),
        ] + [pltpu.SemaphoreType.DMA] * INNER + [
            pltpu.VMEM_SHARED((npad, c), F32),
        ],
        compiler_params=pltpu.CompilerParams(use_tc_tiling_on_sc=False),
    )
    def k(u_hbm, src_hbm, dst_hbm, z_hbm, out_hbm,
          src_v, dst_v, rows_v, *rest):
        sems = rest[:INNER]
        acc = rest[INNER]
        ci = lax.axis_index("c")
        si = lax.axis_index("s")
        tid = ci * ns + si
        pltpu.sync_copy(z_hbm.at[pl.ds(si * rpt, rpt)],
                        acc.at[pl.ds(si * rpt, rpt)])
        plsc.subcore_barrier()
        g0 = tid * ng
        pltpu.sync_copy(src_hbm.at[pl.ds(g0, ng)], src_v)
        pltpu.sync_copy(dst_hbm.at[pl.ds(g0, ng)], dst_v)
        for b in range(INNER):
            pltpu.async_copy(u_hbm.at[src_v.at[b]], rows_v.at[b], sems[b])

        def obody(o, carry):
            for b in range(INNER):
                g = o * INNER + b
                pltpu.make_async_copy(
                    u_hbm.at[src_v.at[g]], rows_v.at[b], sems[b]).wait()
                pltpu.sync_copy(rows_v.at[b], acc.at[dst_v.at[g]], add=True)

                @pl.when(o < outer - 1)
                def _fire():
                    pltpu.async_copy(
                        u_hbm.at[src_v.at[g + INNER]], rows_v.at[b], sems[b])
            return carry

        lax.fori_loop(0, outer, obody, 0)
        plsc.subcore_barrier()
        pltpu.sync_copy(acc.at[pl.ds(si * rpt, rpt)],
                        out_hbm.at[ci, pl.ds(si * rpt, rpt)])

    return k(u, src3, dst3, zeros)


def _deg3(dst3s, npads, widths):
    nc, ns = _sc_geom()
    nw = nc * ns
    ngs = [d.shape[0] // nw for d in dst3s]
    rpts = [np_ // ns for np_ in npads]
    ones = [jnp.ones((G, w), F32) for w in widths]
    zeros = [jnp.zeros((np_, w), F32) for np_, w in zip(npads, widths)]
    mesh = plsc.VectorSubcoreMesh(
        core_axis_name="c", subcore_axis_name="s",
        num_cores=nc, num_subcores=ns)

    @functools.partial(
        pl.kernel,
        out_type=[jax.ShapeDtypeStruct((nc, np_, w), F32)
                  for np_, w in zip(npads, widths)],
        mesh=mesh,
        scratch_types=[pltpu.VMEM((ng, G), I32) for ng in ngs]
        + [pltpu.VMEM((G, w), F32) for w in widths]
        + [pltpu.VMEM_SHARED((np_, w), F32)
           for np_, w in zip(npads, widths)],
        compiler_params=pltpu.CompilerParams(use_tc_tiling_on_sc=False),
    )
    def k(d0_hbm, d1_hbm, d2_hbm, n0_hbm, n1_hbm, n2_hbm,
          z0_hbm, z1_hbm, z2_hbm, o0_hbm, o1_hbm, o2_hbm,
          v0, v1, v2, w0_v, w1_v, w2_v, a0, a1, a2):
        dhbm, dv, accs = [d0_hbm, d1_hbm, d2_hbm], [v0, v1, v2], [a0, a1, a2]
        outs, zhbm = [o0_hbm, o1_hbm, o2_hbm], [z0_hbm, z1_hbm, z2_hbm]
        nhbm, onev = [n0_hbm, n1_hbm, n2_hbm], [w0_v, w1_v, w2_v]
        ci = lax.axis_index("c")
        si = lax.axis_index("s")
        tid = ci * ns + si
        for l in range(3):
            pltpu.sync_copy(zhbm[l].at[pl.ds(si * rpts[l], rpts[l])],
                            accs[l].at[pl.ds(si * rpts[l], rpts[l])])
        plsc.subcore_barrier()
        for l in range(3):
            pltpu.sync_copy(nhbm[l].at[pl.ds(0, G)],
                            onev[l].at[pl.ds(0, G)])
            pltpu.sync_copy(dhbm[l].at[pl.ds(tid * ngs[l], ngs[l])], dv[l])

            def obody(g, carry, l=l):
                pltpu.sync_copy(onev[l], accs[l].at[dv[l].at[g]], add=True)
                return carry

            lax.fori_loop(0, ngs[l], obody, 0)
        plsc.subcore_barrier()
        for l in range(3):
            pltpu.sync_copy(accs[l].at[pl.ds(si * rpts[l], rpts[l])],
                            outs[l].at[ci, pl.ds(si * rpts[l], rpts[l])])

    return k(dst3s[0], dst3s[1], dst3s[2], *ones, *zeros)


_BLK = 512


def _flat(a):
    nc_, npad, c = a.shape
    return a.reshape(nc_, npad * c // 128, 128)


def _prep(p0, p1, a0, a1, scale):
    r = a0.shape[0]
    nb = -(-r // _BLK)

    def body(p0_r, p1_r, a0_r, a1_r, dis_r, t0_r, u0_r):
        deg = p0_r[...] + p1_r[...]
        dis = jnp.where(deg > 0, lax.rsqrt(deg), 0.0)
        t0 = scale * (a0_r[...] + a1_r[...])
        dis_r[...] = dis
        t0_r[...] = t0
        u0_r[...] = dis * t0

    spec = pl.BlockSpec((_BLK, 128), lambda i: (i, 0))
    return pl.pallas_call(
        body,
        grid=(nb,),
        in_specs=[spec] * 4,
        out_specs=[spec] * 3,
        out_shape=[jax.ShapeDtypeStruct((r, 128), F32)] * 3,
    )(p0, p1, a0, a1)


def _combine(s0, s1, dis, tprev, alpha, beta):
    r = s0.shape[0]
    nb = -(-r // _BLK)

    def body(s0_r, s1_r, dis_r, tp_r, t_r, u_r):
        t = alpha * (dis_r[...] * (s0_r[...] + s1_r[...])) + beta * tp_r[...]
        t_r[...] = t
        u_r[...] = dis_r[...] * t

    spec = pl.BlockSpec((_BLK, 128), lambda i: (i, 0))
    return pl.pallas_call(
        body,
        grid=(nb,),
        in_specs=[spec] * 4,
        out_specs=[spec] * 2,
        out_shape=[jax.ShapeDtypeStruct((r, 128), F32)] * 2,
    )(s0, s1, dis, tprev)


def _cheb_matmul(ts, w, b, c, relu):
    r = ts[0].shape[0]
    kk, cout = w.shape[0], w.shape[2]
    m = 128 // c
    eye = jnp.eye(m, dtype=F32)
    wbig = jnp.stack([jnp.kron(eye, w[k]) for k in range(kk)])
    mo = m * cout
    btile = jnp.tile(b, (m,)).reshape(1, mo)
    nb = -(-r // _BLK)

    def body(*refs):
        t_refs = refs[:kk]
        w_r, b_r, h_r = refs[kk], refs[kk + 1], refs[kk + 2]
        acc = b_r[...] + jnp.zeros((_BLK, mo), F32)
        for k in range(kk):
            acc = acc + jnp.dot(t_refs[k][...], w_r[k],
                                preferred_element_type=F32)
        if relu:
            acc = jnp.maximum(acc, 0.0)
        h_r[...] = acc

    return pl.pallas_call(
        body,
        grid=(nb,),
        in_specs=[pl.BlockSpec((_BLK, 128), lambda i: (i, 0))
                  for _ in range(kk)] + [
            pl.BlockSpec((kk, 128, mo), lambda i: (0, 0, 0)),
            pl.BlockSpec((1, mo), lambda i: (0, 0)),
        ],
        out_specs=pl.BlockSpec((_BLK, mo), lambda i: (i, 0)),
        out_shape=jax.ShapeDtypeStruct((r, mo), F32),
    )(*ts, wbig, btile)


def _matvec(wlin, hflat, blin):
    ncls, kdim = wlin.shape
    cb = 8
    ni = ncls // cb

    def body(h_r, w_r, b_r, z_r):
        z_r[...] = b_r[...] + lax.dot_general(
            w_r[...], h_r[...], (((1,), (1,)), ((), ())),
            preferred_element_type=F32)

    out = pl.pallas_call(
        body,
        grid=(ni,),
        in_specs=[
            pl.BlockSpec((1, kdim), lambda i: (0, 0)),
            pl.BlockSpec((cb, kdim), lambda i: (i, 0)),
            pl.BlockSpec((cb, 1), lambda i: (i, 0)),
        ],
        out_specs=pl.BlockSpec((cb, 1), lambda i: (i, 0)),
        out_shape=jax.ShapeDtypeStruct((ncls, 1), F32),
    )(hflat.reshape(1, kdim), wlin, blin.reshape(ncls, 1))
    return out.reshape(ncls)


def _cheb_layer(a0f, a1f, scale, src3, dst3, degf, w, b, n, c, relu):
    nc, ns = _sc_geom()
    npad = _npad(n, ns)
    disf, t0f, uf = _prep(degf[0], degf[1], a0f, a1f, scale)
    ts = [t0f]
    kk = w.shape[0]
    for k in range(1, kk):
        sp = _segsum(uf.reshape(npad, c), src3, dst3, npad)
        spf = _flat(sp)
        alpha, beta = (-1.0, 0.0) if k == 1 else (-2.0, -1.0)
        tprev = ts[k - 2] if k >= 2 else t0f
        t, uf = _combine(spf[0], spf[1], disf, tprev, alpha, beta)
        ts.append(t)
    return _cheb_matmul(ts, w, b, c, relu)


def _pool(hf, npad_in, c, rows, cols, n_out):
    nc, ns = _sc_geom()
    src3, dst3 = _pad_edges(cols, rows, n_out, nc * ns)
    npad_o = _npad(n_out, ns)
    sp = _segsum(hf.reshape(npad_in, c), src3, dst3, npad_o)
    spf = _flat(sp)
    return spf[0], spf[1]


def kernel(x, ei0, ei1, ei2, d0_rows, d0_cols, d0_vals,
           d1_rows, d1_cols, d1_vals,
           W0, b0, W1, b1, W2, b2, Wlin, blin):
    n0, n1, n2 = 50000, 12500, 3125
    nc, ns = _sc_geom()
    nw = nc * ns
    npads = [_npad(n, ns) for n in (n0, n1, n2)]
    x8 = jnp.pad(x, ((0, npads[0] - n0), (0, 5)))
    x8f = x8.reshape(npads[0] * 8 // 128, 128)
    w0p = jnp.pad(W0, ((0, 0), (0, 5), (0, 0)))
    s0, d0 = _pad_edges(ei0[0], ei0[1], n0, nw)
    s1, d1 = _pad_edges(ei1[0], ei1[1], n1, nw)
    s2, d2 = _pad_edges(ei2[0], ei2[1], n2, nw)
    degs = _deg3([d0, d1, d2], npads, [8, 16, 32])
    degf = [_flat(g) for g in degs]
    h0f = _cheb_layer(x8f, x8f, 0.5, s0, d0, degf[0], w0p, b0,
                      n0, 8, relu=True)
    a0f, a1f = _pool(h0f, npads[0], 16, d0_rows, d0_cols, n1)
    h1f = _cheb_layer(a0f, a1f, 0.25, s1, d1, degf[1], W1, b1,
                      n1, 16, relu=True)
    a0f, a1f = _pool(h1f, npads[1], 32, d1_rows, d1_cols, n2)
    h2f = _cheb_layer(a0f, a1f, 0.25, s2, d2, degf[2], W2, b2,
                      n2, 32, relu=False)
    h2 = h2f.reshape(npads[2], 32)[:n2]
    return _matvec(Wlin, h2.reshape(-1), blin)


# speedup vs baseline: 1.4896x; 1.0063x over previous
import functools

import jax
import jax.numpy as jnp
from jax import lax
from jax.experimental import pallas as pl
from jax.experimental.pallas import tpu as pltpu
from jax.experimental.pallas import tpu_sc as plsc

F32 = jnp.float32
I32 = jnp.int32
G = 128
INNER = 4


def _sc_geom():
    try:
        info = plsc.get_sparse_core_info()
        return int(info.num_cores), int(info.num_subcores)
    except Exception:
        return 2, 16


def _npad(n, ns):
    q = 8 * ns
    return q * (-(-(n + 1) // q))


def _pad_edges(src, dst, n, nw):
    e = src.shape[0]
    quantum = nw * G * INNER
    epad = quantum * (-(-e // quantum))
    pad = epad - e
    src_p = jnp.concatenate([src, jnp.zeros((pad,), I32)])
    dst_p = jnp.concatenate([dst, jnp.full((pad,), n, I32)])
    return src_p.reshape(-1, G), dst_p.reshape(-1, G)


def _segsum(u, src3, dst3, npad):
    n, c = u.shape
    nc, ns = _sc_geom()
    nw = nc * ns
    ng = src3.shape[0] // nw
    outer = ng // INNER
    rpt = npad // ns
    zeros = jnp.zeros((npad, c), F32)
    mesh = plsc.VectorSubcoreMesh(
        core_axis_name="c", subcore_axis_name="s",
        num_cores=nc, num_subcores=ns)

    @functools.partial(
        pl.kernel,
        out_type=jax.ShapeDtypeStruct((nc, npad, c), F32),
        mesh=mesh,
        scratch_types=[
            pltpu.VMEM((ng, G), I32),
            pltpu.VMEM((ng, G), I32),
            pltpu.VMEM((INNER, G, c), F32),
        ] + [pltpu.SemaphoreType.DMA] * INNER + [
            pltpu.VMEM_SHARED((npad, c), F32),
        ],
        compiler_params=pltpu.CompilerParams(use_tc_tiling_on_sc=False),
    )
    def k(u_hbm, src_hbm, dst_hbm, z_hbm, out_hbm,
          src_v, dst_v, rows_v, *rest):
        sems = rest[:INNER]
        acc = rest[INNER]
        ci = lax.axis_index("c")
        si = lax.axis_index("s")
        tid = ci * ns + si
        g0 = tid * ng
        pltpu.sync_copy(src_hbm.at[pl.ds(g0, ng)], src_v)
        pltpu.sync_copy(dst_hbm.at[pl.ds(g0, ng)], dst_v)
        for b in range(INNER):
            pltpu.async_copy(u_hbm.at[src_v.at[b]], rows_v.at[b], sems[b])
        pltpu.sync_copy(z_hbm.at[pl.ds(si * rpt, rpt)],
                        acc.at[pl.ds(si * rpt, rpt)])
        plsc.subcore_barrier()

        def obody(o, carry):
            for b in range(INNER):
                g = o * INNER + b
                pltpu.make_async_copy(
                    u_hbm.at[src_v.at[g]], rows_v.at[b], sems[b]).wait()
                pltpu.sync_copy(rows_v.at[b], acc.at[dst_v.at[g]], add=True)

                @pl.when(o < outer - 1)
                def _fire():
                    pltpu.async_copy(
                        u_hbm.at[src_v.at[g + INNER]], rows_v.at[b], sems[b])
            return carry

        lax.fori_loop(0, outer, obody, 0)
        plsc.subcore_barrier()
        pltpu.sync_copy(acc.at[pl.ds(si * rpt, rpt)],
                        out_hbm.at[ci, pl.ds(si * rpt, rpt)])

    return k(u, src3, dst3, zeros)


def _deg3(dst3s, npads, widths):
    nc, ns = _sc_geom()
    nw = nc * ns
    ngs = [d.shape[0] // nw for d in dst3s]
    rpts = [np_ // ns for np_ in npads]
    ones = [jnp.ones((G, w), F32) for w in widths]
    zeros = [jnp.zeros((np_, w), F32) for np_, w in zip(npads, widths)]
    mesh = plsc.VectorSubcoreMesh(
        core_axis_name="c", subcore_axis_name="s",
        num_cores=nc, num_subcores=ns)

    @functools.partial(
        pl.kernel,
        out_type=[jax.ShapeDtypeStruct((nc, np_, w), F32)
                  for np_, w in zip(npads, widths)],
        mesh=mesh,
        scratch_types=[pltpu.VMEM((ng, G), I32) for ng in ngs]
        + [pltpu.VMEM((G, w), F32) for w in widths]
        + [pltpu.VMEM_SHARED((np_, w), F32)
           for np_, w in zip(npads, widths)],
        compiler_params=pltpu.CompilerParams(use_tc_tiling_on_sc=False),
    )
    def k(d0_hbm, d1_hbm, d2_hbm, n0_hbm, n1_hbm, n2_hbm,
          z0_hbm, z1_hbm, z2_hbm, o0_hbm, o1_hbm, o2_hbm,
          v0, v1, v2, w0_v, w1_v, w2_v, a0, a1, a2):
        dhbm, dv, accs = [d0_hbm, d1_hbm, d2_hbm], [v0, v1, v2], [a0, a1, a2]
        outs, zhbm = [o0_hbm, o1_hbm, o2_hbm], [z0_hbm, z1_hbm, z2_hbm]
        nhbm, onev = [n0_hbm, n1_hbm, n2_hbm], [w0_v, w1_v, w2_v]
        ci = lax.axis_index("c")
        si = lax.axis_index("s")
        tid = ci * ns + si
        for l in range(3):
            pltpu.sync_copy(zhbm[l].at[pl.ds(si * rpts[l], rpts[l])],
                            accs[l].at[pl.ds(si * rpts[l], rpts[l])])
        plsc.subcore_barrier()
        for l in range(3):
            pltpu.sync_copy(nhbm[l].at[pl.ds(0, G)],
                            onev[l].at[pl.ds(0, G)])
            pltpu.sync_copy(dhbm[l].at[pl.ds(tid * ngs[l], ngs[l])], dv[l])

            def obody(g, carry, l=l):
                pltpu.sync_copy(onev[l], accs[l].at[dv[l].at[g]], add=True)
                return carry

            lax.fori_loop(0, ngs[l], obody, 0)
        plsc.subcore_barrier()
        for l in range(3):
            pltpu.sync_copy(accs[l].at[pl.ds(si * rpts[l], rpts[l])],
                            outs[l].at[ci, pl.ds(si * rpts[l], rpts[l])])

    return k(dst3s[0], dst3s[1], dst3s[2], *ones, *zeros)


_BLK = 512


def _flat(a):
    nc_, npad, c = a.shape
    return a.reshape(nc_, npad * c // 128, 128)


def _prep(p0, p1, a0, a1, scale):
    r = a0.shape[0]
    nb = -(-r // _BLK)

    def body(p0_r, p1_r, a0_r, a1_r, dis_r, t0_r, u0_r):
        deg = p0_r[...] + p1_r[...]
        dis = jnp.where(deg > 0, lax.rsqrt(deg), 0.0)
        t0 = scale * (a0_r[...] + a1_r[...])
        dis_r[...] = dis
        t0_r[...] = t0
        u0_r[...] = dis * t0

    spec = pl.BlockSpec((_BLK, 128), lambda i: (i, 0))
    return pl.pallas_call(
        body,
        grid=(nb,),
        in_specs=[spec] * 4,
        out_specs=[spec] * 3,
        out_shape=[jax.ShapeDtypeStruct((r, 128), F32)] * 3,
    )(p0, p1, a0, a1)


def _combine(s0, s1, dis, tprev, alpha, beta):
    r = s0.shape[0]
    nb = -(-r // _BLK)

    def body(s0_r, s1_r, dis_r, tp_r, t_r, u_r):
        t = alpha * (dis_r[...] * (s0_r[...] + s1_r[...])) + beta * tp_r[...]
        t_r[...] = t
        u_r[...] = dis_r[...] * t

    spec = pl.BlockSpec((_BLK, 128), lambda i: (i, 0))
    return pl.pallas_call(
        body,
        grid=(nb,),
        in_specs=[spec] * 4,
        out_specs=[spec] * 2,
        out_shape=[jax.ShapeDtypeStruct((r, 128), F32)] * 2,
    )(s0, s1, dis, tprev)


def _cheb_matmul(ts, w, b, c, relu):
    r = ts[0].shape[0]
    kk, cout = w.shape[0], w.shape[2]
    m = 128 // c
    eye = jnp.eye(m, dtype=F32)
    wbig = jnp.stack([jnp.kron(eye, w[k]) for k in range(kk)])
    mo = m * cout
    btile = jnp.tile(b, (m,)).reshape(1, mo)
    nb = -(-r // _BLK)

    def body(*refs):
        t_refs = refs[:kk]
        w_r, b_r, h_r = refs[kk], refs[kk + 1], refs[kk + 2]
        acc = b_r[...] + jnp.zeros((_BLK, mo), F32)
        for k in range(kk):
            acc = acc + jnp.dot(t_refs[k][...], w_r[k],
                                preferred_element_type=F32)
        if relu:
            acc = jnp.maximum(acc, 0.0)
        h_r[...] = acc

    return pl.pallas_call(
        body,
        grid=(nb,),
        in_specs=[pl.BlockSpec((_BLK, 128), lambda i: (i, 0))
                  for _ in range(kk)] + [
            pl.BlockSpec((kk, 128, mo), lambda i: (0, 0, 0)),
            pl.BlockSpec((1, mo), lambda i: (0, 0)),
        ],
        out_specs=pl.BlockSpec((_BLK, mo), lambda i: (i, 0)),
        out_shape=jax.ShapeDtypeStruct((r, mo), F32),
    )(*ts, wbig, btile)


def _matvec(wlin, hflat, blin):
    ncls, kdim = wlin.shape
    cb = 8
    ni = ncls // cb

    def body(h_r, w_r, b_r, z_r):
        z_r[...] = b_r[...] + lax.dot_general(
            w_r[...], h_r[...], (((1,), (1,)), ((), ())),
            preferred_element_type=F32)

    out = pl.pallas_call(
        body,
        grid=(ni,),
        in_specs=[
            pl.BlockSpec((1, kdim), lambda i: (0, 0)),
            pl.BlockSpec((cb, kdim), lambda i: (i, 0)),
            pl.BlockSpec((cb, 1), lambda i: (i, 0)),
        ],
        out_specs=pl.BlockSpec((cb, 1), lambda i: (i, 0)),
        out_shape=jax.ShapeDtypeStruct((ncls, 1), F32),
    )(hflat.reshape(1, kdim), wlin, blin.reshape(ncls, 1))
    return out.reshape(ncls)


def _cheb_layer(a0f, a1f, scale, src3, dst3, degf, w, b, n, c, relu):
    nc, ns = _sc_geom()
    npad = _npad(n, ns)
    disf, t0f, uf = _prep(degf[0], degf[1], a0f, a1f, scale)
    ts = [t0f]
    kk = w.shape[0]
    for k in range(1, kk):
        sp = _segsum(uf.reshape(npad, c), src3, dst3, npad)
        spf = _flat(sp)
        alpha, beta = (-1.0, 0.0) if k == 1 else (-2.0, -1.0)
        tprev = ts[k - 2] if k >= 2 else t0f
        t, uf = _combine(spf[0], spf[1], disf, tprev, alpha, beta)
        ts.append(t)
    return _cheb_matmul(ts, w, b, c, relu)


def _pool(hf, npad_in, c, rows, cols, n_out):
    nc, ns = _sc_geom()
    src3, dst3 = _pad_edges(cols, rows, n_out, nc * ns)
    npad_o = _npad(n_out, ns)
    sp = _segsum(hf.reshape(npad_in, c), src3, dst3, npad_o)
    spf = _flat(sp)
    return spf[0], spf[1]


def kernel(x, ei0, ei1, ei2, d0_rows, d0_cols, d0_vals,
           d1_rows, d1_cols, d1_vals,
           W0, b0, W1, b1, W2, b2, Wlin, blin):
    n0, n1, n2 = 50000, 12500, 3125
    nc, ns = _sc_geom()
    nw = nc * ns
    npads = [_npad(n, ns) for n in (n0, n1, n2)]
    x8 = jnp.pad(x, ((0, npads[0] - n0), (0, 5)))
    x8f = x8.reshape(npads[0] * 8 // 128, 128)
    w0p = jnp.pad(W0, ((0, 0), (0, 5), (0, 0)))
    s0, d0 = _pad_edges(ei0[0], ei0[1], n0, nw)
    s1, d1 = _pad_edges(ei1[0], ei1[1], n1, nw)
    s2, d2 = _pad_edges(ei2[0], ei2[1], n2, nw)
    degs = _deg3([d0, d1, d2], npads, [8, 16, 32])
    degf = [_flat(g) for g in degs]
    h0f = _cheb_layer(x8f, x8f, 0.5, s0, d0, degf[0], w0p, b0,
                      n0, 8, relu=True)
    a0f, a1f = _pool(h0f, npads[0], 16, d0_rows, d0_cols, n1)
    h1f = _cheb_layer(a0f, a1f, 0.25, s1, d1, degf[1], W1, b1,
                      n1, 16, relu=True)
    a0f, a1f = _pool(h1f, npads[1], 32, d1_rows, d1_cols, n2)
    h2f = _cheb_layer(a0f, a1f, 0.25, s2, d2, degf[2], W2, b2,
                      n2, 32, relu=False)
    h2 = h2f.reshape(npads[2], 32)[:n2]
    return _matvec(Wlin, h2.reshape(-1), blin)
